# baseline (speedup 1.0000x reference)
import functools

import jax
import jax.numpy as jnp
from jax import lax
from jax.experimental import pallas as pl
from jax.experimental.pallas import tpu as pltpu
from jax.experimental.pallas import tpu_sc as plsc

_CR = 2048


@functools.lru_cache(maxsize=None)
def _make_sc_call(B, D, G):
    info = plsc.get_sparse_core_info()
    NC, NS, L = info.num_cores, info.num_subcores, info.num_lanes
    NW = NC * NS
    BW = B // NW
    CR = _CR
    NCHUNK = BW // CR
    GROUPS = CR // L
    assert B % (NW * CR) == 0 and CR % L == 0 and NCHUNK % 2 == 0

    TABN = (G - 1) * (G - 1) + 1
    TABN = ((TABN + L - 1) // L) * L

    mesh = plsc.VectorSubcoreMesh(core_axis_name="c", subcore_axis_name="s")

    @functools.partial(
        pl.kernel,
        mesh=mesh,
        out_type=jax.ShapeDtypeStruct((B,), jnp.float32),
        compiler_params=pltpu.CompilerParams(needs_layout_passes=False),
        scratch_types=[
            pltpu.VMEM((4, CR), jnp.float32),
            pltpu.VMEM((4, CR), jnp.float32),
            pltpu.VMEM((4, CR), jnp.float32),
            pltpu.VMEM((4, CR), jnp.float32),
            pltpu.VMEM((G, CR), jnp.float32),
            pltpu.VMEM((G, CR), jnp.float32),
            pltpu.VMEM((CR,), jnp.float32),
            pltpu.VMEM((CR,), jnp.float32),
            pltpu.VMEM((TABN,), jnp.int32),
            pltpu.SemaphoreType.DMA,
            pltpu.SemaphoreType.DMA,
            pltpu.SemaphoreType.DMA,
            pltpu.SemaphoreType.DMA,
        ],
    )
    def sc_kernel(z1_hbm, z2_hbm, dg_hbm, tab_hbm, out_hbm,
                  z1v0, z1v1, z2v0, z2v1, dgv0, dgv1, outv0, outv1, tabv,
                  semi0, semi1, semo0, semo1):
        z1s, z2s, dgs, outs = [z1v0, z1v1], [z2v0, z2v1], [dgv0, dgv1], [outv0, outv1]
        semis, semos = [semi0, semi1], [semo0, semo1]
        wid = lax.axis_index("s") * NC + lax.axis_index("c")
        row0 = wid * BW
        lanes = lax.iota(jnp.int32, L)

        pltpu.sync_copy(tab_hbm, tabv)

        def start_in(ci, b):
            base = row0 + ci * CR
            pltpu.async_copy(
                z1_hbm.at[pl.ds(0, 4), pl.ds(base, CR)], z1s[b], semis[b])
            pltpu.async_copy(
                z2_hbm.at[pl.ds(0, 4), pl.ds(base, CR)], z2s[b], semis[b])
            pltpu.async_copy(
                dg_hbm.at[:, pl.ds(base, CR)], dgs[b], semis[b])

        def wait_in(b):
            pltpu.make_async_copy(
                z1_hbm.at[pl.ds(0, 4), pl.ds(0, CR)], z1s[b], semis[b]).wait()
            pltpu.make_async_copy(
                z2_hbm.at[pl.ds(0, 4), pl.ds(0, CR)], z2s[b], semis[b]).wait()
            pltpu.make_async_copy(
                dg_hbm.at[:, pl.ds(0, CR)], dgs[b], semis[b]).wait()

        def compute(b):
            z1b, z2b, dgb, outb = z1s[b], z2s[b], dgs[b], outs[b]

            def group_body(g, c_):
                off = g * L
                a0 = z1b[0, pl.ds(off, L)]
                a1 = z1b[1, pl.ds(off, L)]
                a2 = z1b[2, pl.ds(off, L)]
                a3 = z1b[3, pl.ds(off, L)]
                b0 = z2b[0, pl.ds(off, L)]
                b1 = z2b[1, pl.ds(off, L)]
                b2 = z2b[2, pl.ds(off, L)]
                b3 = z2b[3, pl.ds(off, L)]
                dx = (b0 + b2) - (a0 + a2)
                dy = (b1 + b3) - (a1 + a3)
                r2 = dx * dx + dy * dy
                q = jnp.minimum((r2 * 16.0).astype(jnp.int32), TABN - 1)
                did = plsc.load_gather(tabv, [q])
                outb[pl.ds(off, L)] = plsc.load_gather(dgb, [did, lanes + off])
                return c_

            lax.fori_loop(0, GROUPS, group_body, 0, unroll=8)

        def start_out(ci, b):
            base = row0 + ci * CR
            pltpu.async_copy(outs[b], out_hbm.at[pl.ds(base, CR)], semos[b])

        def wait_out(b):
            pltpu.make_async_copy(
                outs[b], out_hbm.at[pl.ds(0, CR)], semos[b]).wait()

        start_in(0, 0)

        def loop_body(ci2, carry):
            for b in range(2):
                ci = ci2 * 2 + b

                @pl.when(ci + 1 < NCHUNK)
                def _():
                    start_in(ci + 1, (b + 1) % 2)

                wait_in(b)

                @pl.when(ci >= 2)
                def _():
                    wait_out(b)

                compute(b)
                start_out(ci, b)
            return carry

        lax.fori_loop(0, NCHUNK // 2, loop_body, 0)
        wait_out(0)
        wait_out(1)

    return sc_kernel


def kernel(z_1, z_2, dist_grade):
    B, D = z_1.shape
    G = dist_grade.shape[1]
    L = plsc.get_sparse_core_info().num_lanes
    tabn = (G - 1) * (G - 1) + 1
    tabn = ((tabn + L - 1) // L) * L
    tab = jnp.asarray(
        [min(int(i ** 0.5), G - 1) for i in range(tabn)], dtype=jnp.int32)
    call = _make_sc_call(B, D, G)
    return call(z_1.T, z_2.T, dist_grade.T, tab)


# speedup vs baseline: 1.2240x; 1.2240x over previous
import functools

import jax
import jax.numpy as jnp
from jax import lax
from jax.experimental import pallas as pl
from jax.experimental.pallas import tpu as pltpu
from jax.experimental.pallas import tpu_sc as plsc


@functools.lru_cache(maxsize=None)
def _make_sc_call(B, D, G):
    info = plsc.get_sparse_core_info()
    NC, NS, L = info.num_cores, info.num_subcores, info.num_lanes
    NW = NC * NS
    BW = B // NW
    CR = 2048
    NCHUNK = BW // CR
    GROUPS = CR // L
    assert B % (NW * CR) == 0 and CR % L == 0 and NCHUNK % 2 == 0

    thr = [4.0 * j * j / (G * G) for j in range(1, G)]

    mesh = plsc.VectorSubcoreMesh(core_axis_name="c", subcore_axis_name="s")

    @functools.partial(
        pl.kernel,
        mesh=mesh,
        out_type=jax.ShapeDtypeStruct((B,), jnp.float32),
        compiler_params=pltpu.CompilerParams(needs_layout_passes=False),
        scratch_types=[
            pltpu.VMEM((4, CR), jnp.float32),
            pltpu.VMEM((4, CR), jnp.float32),
            pltpu.VMEM((4, CR), jnp.float32),
            pltpu.VMEM((4, CR), jnp.float32),
            pltpu.VMEM((G, CR), jnp.float32),
            pltpu.VMEM((G, CR), jnp.float32),
            pltpu.VMEM((CR,), jnp.float32),
            pltpu.VMEM((CR,), jnp.float32),
            pltpu.SemaphoreType.DMA,
            pltpu.SemaphoreType.DMA,
            pltpu.SemaphoreType.DMA,
            pltpu.SemaphoreType.DMA,
        ],
    )
    def sc_kernel(z1_hbm, z2_hbm, dg_hbm, out_hbm,
                  z1v0, z1v1, z2v0, z2v1, dgv0, dgv1, outv0, outv1,
                  semi0, semi1, semo0, semo1):
        z1s, z2s, dgs, outs = [z1v0, z1v1], [z2v0, z2v1], [dgv0, dgv1], [outv0, outv1]
        semis, semos = [semi0, semi1], [semo0, semo1]
        wid = lax.axis_index("s") * NC + lax.axis_index("c")
        row0 = wid * BW
        lanes = lax.iota(jnp.int32, L)

        def start_in(ci, b):
            base = row0 + ci * CR
            pltpu.async_copy(
                z1_hbm.at[pl.ds(0, 4), pl.ds(base, CR)], z1s[b], semis[b])
            pltpu.async_copy(
                z2_hbm.at[pl.ds(0, 4), pl.ds(base, CR)], z2s[b], semis[b])
            pltpu.async_copy(
                dg_hbm.at[:, pl.ds(base, CR)], dgs[b], semis[b])

        def wait_in(b):
            pltpu.make_async_copy(
                z1_hbm.at[pl.ds(0, 4), pl.ds(0, CR)], z1s[b], semis[b]
            ).wait()
            pltpu.make_async_copy(
                z2_hbm.at[pl.ds(0, 4), pl.ds(0, CR)], z2s[b], semis[b]
            ).wait()
            pltpu.make_async_copy(
                dg_hbm.at[:, pl.ds(0, CR)], dgs[b], semis[b]
            ).wait()

        def compute(b):
            z1b, z2b, dgb, outb = z1s[b], z2s[b], dgs[b], outs[b]

            def group_body(g, c_):
                off = g * L
                a0 = z1b[0, pl.ds(off, L)]
                a1 = z1b[1, pl.ds(off, L)]
                a2 = z1b[2, pl.ds(off, L)]
                a3 = z1b[3, pl.ds(off, L)]
                b0 = z2b[0, pl.ds(off, L)]
                b1 = z2b[1, pl.ds(off, L)]
                b2 = z2b[2, pl.ds(off, L)]
                b3 = z2b[3, pl.ds(off, L)]
                dx = (b0 + b2) - (a0 + a2)
                dy = (b1 + b3) - (a1 + a3)
                r2 = dx * dx + dy * dy
                did = (r2 >= thr[0]).astype(jnp.int32)
                for t in thr[1:]:
                    did = did + (r2 >= t).astype(jnp.int32)
                outb[pl.ds(off, L)] = plsc.load_gather(dgb, [did, lanes + off])
                return c_

            lax.fori_loop(0, GROUPS, group_body, 0, unroll=8)

        def start_out(ci, b):
            base = row0 + ci * CR
            pltpu.async_copy(outs[b], out_hbm.at[pl.ds(base, CR)], semos[b])

        def wait_out(b):
            pltpu.make_async_copy(
                outs[b], out_hbm.at[pl.ds(0, CR)], semos[b]
            ).wait()

        start_in(0, 0)

        def loop_body(ci2, carry):
            for b in range(2):
                ci = ci2 * 2 + b

                @pl.when(ci + 1 < NCHUNK)
                def _():
                    start_in(ci + 1, (b + 1) % 2)

                wait_in(b)

                @pl.when(ci >= 2)
                def _():
                    wait_out(b)

                compute(b)
                start_out(ci, b)
            return carry

        lax.fori_loop(0, NCHUNK // 2, loop_body, 0)
        wait_out(0)
        wait_out(1)

    return sc_kernel


def kernel(z_1, z_2, dist_grade):
    B, D = z_1.shape
    G = dist_grade.shape[1]
    call = _make_sc_call(B, D, G)
    return call(z_1.T, z_2.T, dist_grade.T)


# speedup vs baseline: 1.9337x; 1.5798x over previous
import functools

import jax
import jax.numpy as jnp
from jax import lax
from jax.experimental import pallas as pl
from jax.experimental.pallas import tpu as pltpu
from jax.experimental.pallas import tpu_sc as plsc


@functools.lru_cache(maxsize=None)
def _make_sc_call(B, D, G):
    info = plsc.get_sparse_core_info()
    NC, NS, L = info.num_cores, info.num_subcores, info.num_lanes
    NW = NC * NS
    BW = B // NW
    CR = 2048
    NCHUNK = BW // CR
    GROUPS = CR // L
    assert B % (NW * CR) == 0 and CR % L == 0 and NCHUNK % 2 == 0

    thr = [4.0 * j * j / (G * G) for j in range(1, G)]

    mesh = plsc.VectorSubcoreMesh(core_axis_name="c", subcore_axis_name="s")

    @functools.partial(
        pl.kernel,
        mesh=mesh,
        out_type=jax.ShapeDtypeStruct((B,), jnp.float32),
        compiler_params=pltpu.CompilerParams(needs_layout_passes=False),
        scratch_types=[
            pltpu.VMEM((4, CR), jnp.float32),
            pltpu.VMEM((4, CR), jnp.float32),
            pltpu.VMEM((4, CR), jnp.float32),
            pltpu.VMEM((4, CR), jnp.float32),
            pltpu.VMEM((G, CR), jnp.float32),
            pltpu.VMEM((G, CR), jnp.float32),
            pltpu.VMEM((CR,), jnp.float32),
            pltpu.VMEM((CR,), jnp.float32),
            pltpu.SemaphoreType.DMA,
            pltpu.SemaphoreType.DMA,
            pltpu.SemaphoreType.DMA,
            pltpu.SemaphoreType.DMA,
        ],
    )
    def sc_kernel(z1_hbm, z2_hbm, dg_hbm, out_hbm,
                  z1v0, z1v1, z2v0, z2v1, dgv0, dgv1, outv0, outv1,
                  semi0, semi1, semo0, semo1):
        z1s, z2s, dgs, outs = [z1v0, z1v1], [z2v0, z2v1], [dgv0, dgv1], [outv0, outv1]
        semis, semos = [semi0, semi1], [semo0, semo1]
        wid = lax.axis_index("s") * NC + lax.axis_index("c")
        row0 = wid * BW
        lanes = lax.iota(jnp.int32, L)

        def start_in(ci, b):
            base = row0 + ci * CR
            pltpu.async_copy(
                z1_hbm.at[pl.ds(0, 4), pl.ds(base, CR)], z1s[b], semis[b])
            pltpu.async_copy(
                z2_hbm.at[pl.ds(0, 4), pl.ds(base, CR)], z2s[b], semis[b])
            pltpu.async_copy(
                dg_hbm.at[:, pl.ds(base, CR)], dgs[b], semis[b])

        def wait_in(b):
            pltpu.make_async_copy(
                z1_hbm.at[pl.ds(0, 4), pl.ds(0, CR)], z1s[b], semis[b]
            ).wait()
            pltpu.make_async_copy(
                z2_hbm.at[pl.ds(0, 4), pl.ds(0, CR)], z2s[b], semis[b]
            ).wait()
            pltpu.make_async_copy(
                dg_hbm.at[:, pl.ds(0, CR)], dgs[b], semis[b]
            ).wait()

        def compute(b):
            z1b, z2b, dgb, outb = z1s[b], z2s[b], dgs[b], outs[b]

            @plsc.parallel_loop(0, GROUPS, 1, unroll=4)
            def _(g):
                off = g * L
                a0 = z1b[0, pl.ds(off, L)]
                a1 = z1b[1, pl.ds(off, L)]
                a2 = z1b[2, pl.ds(off, L)]
                a3 = z1b[3, pl.ds(off, L)]
                b0 = z2b[0, pl.ds(off, L)]
                b1 = z2b[1, pl.ds(off, L)]
                b2 = z2b[2, pl.ds(off, L)]
                b3 = z2b[3, pl.ds(off, L)]
                dx = (b0 + b2) - (a0 + a2)
                dy = (b1 + b3) - (a1 + a3)
                r2 = dx * dx + dy * dy
                did = (r2 >= thr[0]).astype(jnp.int32)
                for t in thr[1:]:
                    did = did + (r2 >= t).astype(jnp.int32)
                outb[pl.ds(off, L)] = plsc.load_gather(dgb, [did, lanes + off])

        def start_out(ci, b):
            base = row0 + ci * CR
            pltpu.async_copy(outs[b], out_hbm.at[pl.ds(base, CR)], semos[b])

        def wait_out(b):
            pltpu.make_async_copy(
                outs[b], out_hbm.at[pl.ds(0, CR)], semos[b]
            ).wait()

        start_in(0, 0)

        def loop_body(ci2, carry):
            for b in range(2):
                ci = ci2 * 2 + b

                @pl.when(ci + 1 < NCHUNK)
                def _():
                    start_in(ci + 1, (b + 1) % 2)

                wait_in(b)

                @pl.when(ci >= 2)
                def _():
                    wait_out(b)

                compute(b)
                start_out(ci, b)
            return carry

        lax.fori_loop(0, NCHUNK // 2, loop_body, 0)
        wait_out(0)
        wait_out(1)

    return sc_kernel


def kernel(z_1, z_2, dist_grade):
    B, D = z_1.shape
    G = dist_grade.shape[1]
    call = _make_sc_call(B, D, G)
    return call(z_1.T, z_2.T, dist_grade.T)
